# Initial kernel scaffold; baseline (speedup 1.0000x reference)
import jax
import jax.numpy as jnp
from jax.experimental import pallas as pl


def kernel(node_embeddings, batch_indices, W_ih, W_hh, b_ih, b_hh, W1, b1, W2, b2):
    raise NotImplementedError("write your pallas kernel here")


# speedup vs baseline: 11.2209x; 11.2209x over previous
import functools

import jax
import jax.numpy as jnp
from jax import lax
from jax.experimental import pallas as pl
from jax.experimental.pallas import tpu as pltpu
from jax.experimental.pallas import tpu_sc as plsc

N = 100000
H = 128
OUT = 128
NG = 1024
STEPS = 6

NC = 2
NS = 16
NW = NC * NS
GPW = NG // NW
T = 128
U = 4
F = T + 24
NEG = -1e30
L = 16
KV = H // L


def _splat_f32(x):
    return jnp.broadcast_to(x.astype(jnp.float32), (L,))


def _splat_i32(x):
    return jnp.broadcast_to(jnp.int32(x) if isinstance(x, int) else x.astype(jnp.int32), (L,))


def _extract_i32(ref, idx_scalar):
    v = plsc.load_gather(ref, [jnp.full((L,), idx_scalar, jnp.int32)])
    return jnp.max(v)


def _hsum_splat(acc):
    return jnp.broadcast_to(jnp.sum(acc), (L,))


def _make_readout():
    mesh = plsc.VectorSubcoreMesh(core_axis_name="c", subcore_axis_name="s")

    @functools.partial(
        pl.kernel,
        mesh=mesh,
        compiler_params=pltpu.CompilerParams(needs_layout_passes=False),
        out_type=jax.ShapeDtypeStruct((NG, H), jnp.float32),
        scratch_types=[
            pltpu.VMEM((GPW + L,), jnp.int32),
            pltpu.VMEM((GPW, H), jnp.float32),
            pltpu.VMEM((GPW, H), jnp.float32),
            pltpu.VMEM((F, H), jnp.float32),
        ],
    )
    def readout(ne_hbm, starts_hbm, h_hbm, r_hbm, starts_v, h_v, r_v, buf):
        wid = lax.axis_index("s") * NC + lax.axis_index("c")
        g0 = wid * GPW
        pltpu.sync_copy(starts_hbm.at[pl.ds(g0, GPW + L)], starts_v)
        pltpu.sync_copy(h_hbm.at[pl.ds(g0, GPW)], h_v)

        def graph_body(j, _):
            lo = _extract_i32(starts_v, j)
            hi = _extract_i32(starts_v, j + 1)
            hvecs = [h_v[j, pl.ds(L * k, L)] for k in range(KV)]

            def tile_body(t, carry):
                m_v, s_v = carry[0], carry[1]
                v_list = list(carry[2:])
                ts = lo + t * T
                fb = jnp.minimum(ts, N - F) // 8 * 8
                pltpu.sync_copy(ne_hbm.at[pl.ds(fb, F)], buf)
                rend = jnp.minimum(ts + T, hi)
                nrows = rend - ts
                ngrp = (nrows + (U - 1)) // U
                off = ts - fb
                rend_v = _splat_i32(rend)

                def group_body(g, gc):
                    m_v, s_v = gc[0], gc[1]
                    v_list = list(gc[2:])
                    base = off + g * U
                    rows = []
                    es = []
                    for u in range(U):
                        iloc = jnp.minimum(base + u, F - 1)
                        rvs = [buf[iloc, pl.ds(L * k, L)] for k in range(KV)]
                        acc = rvs[0] * hvecs[0]
                        for k in range(1, KV):
                            acc = acc + rvs[k] * hvecs[k]
                        e_u = _hsum_splat(acc)
                        valid = _splat_i32(ts + g * U + u) < rend_v
                        es.append(jnp.where(valid, e_u, NEG))
                        rows.append(rvs)
                    m_new = m_v
                    for u in range(U):
                        m_new = jnp.maximum(m_new, es[u])
                    alpha = jnp.exp(m_v - m_new)
                    ws = [jnp.exp(es[u] - m_new) for u in range(U)]
                    wsum = ws[0]
                    for u in range(1, U):
                        wsum = wsum + ws[u]
                    s_v = s_v * alpha + wsum
                    new_v = []
                    for k in range(KV):
                        vk = v_list[k] * alpha
                        for u in range(U):
                            vk = vk + ws[u] * rows[u][k]
                        new_v.append(vk)
                    return (m_new, s_v, *new_v)

                return lax.fori_loop(0, ngrp, group_body, (m_v, s_v, *v_list))

            zero = jnp.zeros((L,), jnp.float32)
            init = (jnp.full((L,), NEG, jnp.float32), zero, *([zero] * KV))
            nt = (hi - lo + (T - 1)) // T
            res = lax.fori_loop(0, nt, tile_body, init)
            s_v = res[1]
            v_list = res[2:]
            denom = s_v + jnp.float32(1e-16)
            for k in range(KV):
                r_v[j, pl.ds(L * k, L)] = v_list[k] / denom
            return 0

        lax.fori_loop(0, GPW, graph_body, 0)
        pltpu.sync_copy(r_v, r_hbm.at[pl.ds(g0, GPW)])

    return readout


_READOUT = _make_readout()


def _lstm_body(h_ref, r_ref, c_ref, wh_ref, wr_ref, b_ref, h_out, c_out):
    gates = (
        jnp.dot(h_ref[...], wh_ref[...], preferred_element_type=jnp.float32)
        + jnp.dot(r_ref[...], wr_ref[...], preferred_element_type=jnp.float32)
        + b_ref[...]
    )
    i = jax.nn.sigmoid(gates[:, :H])
    f = jax.nn.sigmoid(gates[:, H:2 * H])
    g = jnp.tanh(gates[:, 2 * H:3 * H])
    o = jax.nn.sigmoid(gates[:, 3 * H:4 * H])
    c_new = f * c_ref[...] + i * g
    h_out[...] = o * jnp.tanh(c_new)
    c_out[...] = c_new


_LSTM = pl.pallas_call(
    _lstm_body,
    out_shape=[
        jax.ShapeDtypeStruct((NG, H), jnp.float32),
        jax.ShapeDtypeStruct((NG, H), jnp.float32),
    ],
)


def _final_body(h_ref, r_ref, c_ref, wh_ref, wr_ref, b_ref,
                w1h_ref, w1r_ref, b1_ref, w2t_ref, b2_ref, out_ref):
    gates = (
        jnp.dot(h_ref[...], wh_ref[...], preferred_element_type=jnp.float32)
        + jnp.dot(r_ref[...], wr_ref[...], preferred_element_type=jnp.float32)
        + b_ref[...]
    )
    i = jax.nn.sigmoid(gates[:, :H])
    f = jax.nn.sigmoid(gates[:, H:2 * H])
    g = jnp.tanh(gates[:, 2 * H:3 * H])
    o = jax.nn.sigmoid(gates[:, 3 * H:4 * H])
    c_new = f * c_ref[...] + i * g
    h_new = o * jnp.tanh(c_new)
    hidden = jax.nn.relu(
        jnp.dot(h_new, w1h_ref[...], preferred_element_type=jnp.float32)
        + jnp.dot(r_ref[...], w1r_ref[...], preferred_element_type=jnp.float32)
        + b1_ref[...]
    )
    out_ref[...] = (
        jnp.dot(hidden, w2t_ref[...], preferred_element_type=jnp.float32)
        + b2_ref[...]
    )


_FINAL = pl.pallas_call(
    _final_body,
    out_shape=jax.ShapeDtypeStruct((NG, OUT), jnp.float32),
)


def kernel(node_embeddings, batch_indices, W_ih, W_hh, b_ih, b_hh, W1, b1, W2, b2):
    bi = batch_indices.astype(jnp.int32)
    starts = jnp.searchsorted(bi, jnp.arange(NG + 1, dtype=jnp.int32)).astype(jnp.int32)
    starts = jnp.concatenate([starts, jnp.full((L - 1,), N, jnp.int32)])
    Wh = (W_ih[:, :H] + W_hh).T
    Wr = W_ih[:, H:].T
    b = (b_ih + b_hh)[None, :]
    W1h = W1[:, :H].T
    W1r = W1[:, H:].T
    b1r = b1[None, :]
    W2T = W2.T
    b2r = b2[None, :]

    h = jnp.zeros((NG, H), jnp.float32)
    c = jnp.zeros((NG, H), jnp.float32)
    out = None
    for step in range(STEPS):
        r = _READOUT(node_embeddings, starts, h)
        if step < STEPS - 1:
            h, c = _LSTM(h, r, c, Wh, Wr, b)
        else:
            out = _FINAL(h, r, c, Wh, Wr, b, W1h, W1r, b1r, W2T, b2r)
    return out
